# Initial kernel scaffold; baseline (speedup 1.0000x reference)
import jax
import jax.numpy as jnp
from jax.experimental import pallas as pl


def kernel(logits, targets, target_lengths):
    raise NotImplementedError("write your pallas kernel here")


# speedup vs baseline: 1.9318x; 1.9318x over previous
import functools

import jax
import jax.numpy as jnp
from jax import lax
from jax.experimental import pallas as pl
from jax.experimental.pallas import tpu as pltpu
from jax.experimental.pallas import tpu_sc as plsc

B, N, C = 8, 20000, 80
MAX_DET = 300
CONF = 0.8
NMS_T = 0.4
CONF_BITS = 1061997773
KEY_MAX = (1 << 22) - 1
CHUNK = 800
NCHUNK = (B * N) // CHUNK
PAD_DET = 384
PAD_COL = 304


def _k1_body(x_ref, pay_ref, key_ref):
    x = x_ref[0]
    cx, cy, w, h = x[:, 0], x[:, 1], x[:, 2], x[:, 3]
    obj = x[:, 4]
    cls = x[:, 5:]
    mx = jnp.max(cls, axis=-1)
    iot = lax.broadcasted_iota(jnp.int32, cls.shape, 1)
    amax = jnp.min(jnp.where(cls == mx[:, None], iot, C), axis=-1)
    score = obj * mx
    sbits = lax.bitcast_convert_type(score, jnp.int32)
    mkey = jnp.clip(sbits - CONF_BITS, 0, KEY_MAX)
    x1 = cx - w / 2.0
    y1 = cy - h / 2.0
    z = jnp.zeros_like(score)
    pay = jnp.stack(
        [x1, y1, x1 + w, y1 + h, score, amax.astype(jnp.float32), z, z], axis=-1)
    pay_ref[0] = pay
    key_ref[0, 0] = mkey


def _k1(lflat3):
    return pl.pallas_call(
        _k1_body,
        grid=(NCHUNK,),
        in_specs=[pl.BlockSpec((1, CHUNK, 5 + C), lambda i: (i, 0, 0))],
        out_specs=[
            pl.BlockSpec((1, CHUNK, 8), lambda i: (i, 0, 0)),
            pl.BlockSpec((1, 1, CHUNK), lambda i: (i, 0, 0)),
        ],
        out_shape=[
            jax.ShapeDtypeStruct((NCHUNK, CHUNK, 8), jnp.float32),
            jax.ShapeDtypeStruct((NCHUNK, 1, CHUNK), jnp.int32),
        ],
    )(lflat3)


def _k2_standin(mkey2d, pay):
    def one(mk, payb):
        valid = mk >= 1
        need = jnp.minimum(jnp.sum(valid.astype(jnp.int32)), MAX_DET)
        masked = jnp.where(valid, mk, -1)
        _, idx = lax.top_k(masked, MAX_DET)
        idx = jnp.where(jnp.arange(MAX_DET) < need, idx, 0)
        idx = jnp.concatenate([idx, jnp.zeros((PAD_DET - MAX_DET,), jnp.int32)])
        rows = jnp.take(payb, idx, axis=0)
        return rows, jnp.full((16,), need, jnp.int32)

    rows, needs = jax.vmap(one)(mkey2d, pay.reshape(B, N, 8))
    return rows, needs


def _k3_body(rows_ref, needs_ref, tgt_ref, tlen_ref,
             pb_ref, plab_ref, ps_ref, tb_ref, tlab_ref, ts_ref, iou_ref):
    rows = rows_ref[:, :PAD_COL, :]
    x1 = rows[:, :, 0]
    y1 = rows[:, :, 1]
    x2 = rows[:, :, 2]
    y2 = rows[:, :, 3]
    scores = rows[:, :MAX_DET, 4]
    labels = rows[:, :MAX_DET, 5]
    need = needs_ref[:, 0:1]
    area = jnp.clip(x2 - x1, 0.0, None) * jnp.clip(y2 - y1, 0.0, None)

    for ci in range(PAD_COL // 8):
        sl = slice(ci * 8, ci * 8 + 8)
        xi1 = x1[:, sl][:, :, None]
        yi1 = y1[:, sl][:, :, None]
        xi2 = x2[:, sl][:, :, None]
        yi2 = y2[:, sl][:, :, None]
        ai = area[:, sl][:, :, None]
        ltx = jnp.maximum(xi1, x1[:, None, :])
        lty = jnp.maximum(yi1, y1[:, None, :])
        rbx = jnp.minimum(xi2, x2[:, None, :])
        rby = jnp.minimum(yi2, y2[:, None, :])
        inter = jnp.clip(rbx - ltx, 0.0, None) * jnp.clip(rby - lty, 0.0, None)
        union = ai + area[:, None, :] - inter
        iou_ref[:, sl, :] = inter / (union + 1e-9)

    col = lax.broadcasted_iota(jnp.int32, (B, PAD_COL), 1)
    keep0 = (col < need).astype(jnp.float32)

    def nms_step(i, keep):
        row = iou_ref[:, pl.ds(i, 1), :][:, 0, :]
        cond = (row > NMS_T) & (keep > 0.0) & (col < i)
        sup = jnp.max(jnp.where(cond, 1.0, 0.0), axis=-1, keepdims=True)
        return jnp.where((col == i) & (sup > 0.0), 0.0, keep)

    keep = lax.fori_loop(0, MAX_DET, nms_step, keep0, unroll=False)
    keepf = keep[:, :MAX_DET]

    pb_ref[...] = jnp.stack(
        [x1[:, :MAX_DET] * keepf, y1[:, :MAX_DET] * keepf,
         x2[:, :MAX_DET] * keepf, y2[:, :MAX_DET] * keepf], axis=-1)
    ps_ref[...] = scores * keepf
    plab_ref[...] = (labels * keepf).astype(jnp.int32)

    tgt = tgt_ref[...]
    tlen = tlen_ref[...]
    tmf = (lax.broadcasted_iota(jnp.int32, tgt.shape[:2], 1) < tlen
           ).astype(jnp.float32)
    tcx = tgt[:, :, 0] * tmf
    tcy = tgt[:, :, 1] * tmf
    tw = tgt[:, :, 2] * tmf
    th = tgt[:, :, 3] * tmf
    tx1 = tcx - tw / 2.0
    ty1 = tcy - th / 2.0
    tb_ref[...] = jnp.stack([tx1, ty1, tx1 + tw, ty1 + th], axis=-1)
    ts_ref[...] = tgt[:, :, 4] * tmf
    tlab_ref[...] = (tgt[:, :, 5] * tmf).astype(jnp.int32)


def _k3(rows, needs, targets, tlen2d):
    T = targets.shape[1]
    return pl.pallas_call(
        _k3_body,
        out_shape=[
            jax.ShapeDtypeStruct((B, MAX_DET, 4), jnp.float32),
            jax.ShapeDtypeStruct((B, MAX_DET), jnp.int32),
            jax.ShapeDtypeStruct((B, MAX_DET), jnp.float32),
            jax.ShapeDtypeStruct((B, T, 4), jnp.float32),
            jax.ShapeDtypeStruct((B, T), jnp.int32),
            jax.ShapeDtypeStruct((B, T), jnp.float32),
        ],
        scratch_shapes=[pltpu.VMEM((B, PAD_COL, PAD_COL), jnp.float32)],
    )(rows, needs, targets, tlen2d)


def kernel(logits, targets, target_lengths):
    lflat3 = logits.reshape(NCHUNK, CHUNK, 5 + C)
    pay3, mkey3 = _k1(lflat3)
    pay = pay3.reshape(B * N, 8)
    mkey2d = mkey3.reshape(B, N)
    rows, needs = _k2_standin(mkey2d, pay)
    tlen2d = target_lengths.reshape(B, 1)
    return tuple(_k3(rows, needs, targets, tlen2d))


# speedup vs baseline: 1.9663x; 1.0179x over previous
import functools

import jax
import jax.numpy as jnp
from jax import lax
from jax.experimental import pallas as pl
from jax.experimental.pallas import tpu as pltpu
from jax.experimental.pallas import tpu_sc as plsc

B, N, C = 8, 20000, 80
MAX_DET = 300
CONF = 0.8
NMS_T = 0.4
CONF_BITS = 1061997773
KEY_MAX = (1 << 22) - 1
CHUNK = 800
NCHUNK = (B * N) // CHUNK
PAD_DET = 384
PAD_COL = 304


def _k1_body(x_ref, pay_ref, key_ref):
    x = x_ref[0]
    cx, cy, w, h = x[:, 0], x[:, 1], x[:, 2], x[:, 3]
    obj = x[:, 4]
    cls = x[:, 5:]
    mx = jnp.max(cls, axis=-1)
    iot = lax.broadcasted_iota(jnp.int32, cls.shape, 1)
    amax = jnp.min(jnp.where(cls == mx[:, None], iot, C), axis=-1)
    score = obj * mx
    sbits = lax.bitcast_convert_type(score, jnp.int32)
    mkey = jnp.clip(sbits - CONF_BITS, 0, KEY_MAX)
    x1 = cx - w / 2.0
    y1 = cy - h / 2.0
    z = jnp.zeros_like(score)
    pay = jnp.stack(
        [x1, y1, x1 + w, y1 + h, score, amax.astype(jnp.float32), z, z], axis=-1)
    pay_ref[0] = pay
    key_ref[0, 0] = mkey


def _k1(lflat3):
    return pl.pallas_call(
        _k1_body,
        grid=(NCHUNK,),
        in_specs=[pl.BlockSpec((1, CHUNK, 5 + C), lambda i: (i, 0, 0))],
        out_specs=[
            pl.BlockSpec((1, CHUNK, 8), lambda i: (i, 0, 0)),
            pl.BlockSpec((1, 1, CHUNK), lambda i: (i, 0, 0)),
        ],
        out_shape=[
            jax.ShapeDtypeStruct((NCHUNK, CHUNK, 8), jnp.float32),
            jax.ShapeDtypeStruct((NCHUNK, 1, CHUNK), jnp.int32),
        ],
    )(lflat3)


NBLK = N // 16
HSIZE = 2064

_sc_mesh = plsc.VectorSubcoreMesh(core_axis_name="c", subcore_axis_name="s")


@functools.partial(
    pl.kernel,
    mesh=_sc_mesh,
    out_type=[
        jax.ShapeDtypeStruct((B, PAD_DET * 8), jnp.float32),
        jax.ShapeDtypeStruct((B, 16), jnp.int32),
    ],
    scratch_types=[
        pltpu.VMEM((N,), jnp.int32),
        pltpu.VMEM((HSIZE,), jnp.int32),
        pltpu.VMEM((2048,), jnp.int32),
        pltpu.VMEM((PAD_DET,), jnp.int32),
        pltpu.VMEM((PAD_DET,), jnp.int32),
        pltpu.VMEM((PAD_DET,), jnp.int32),
        pltpu.VMEM((PAD_DET,), jnp.int32),
        pltpu.VMEM((PAD_DET // 16, 128), jnp.int32),
        pltpu.VMEM((PAD_DET * 8,), jnp.float32),
        pltpu.VMEM((16,), jnp.int32),
        pltpu.SemaphoreType.DMA,
    ],
    compiler_params=pltpu.CompilerParams(needs_layout_passes=False),
)
def _k2_sc(mk_hbm, pay_hbm, rows_out, needs_out,
           mk_v, hist_v, suf_v, gt_v, eq_v, keys_v, srt_v, gidx_v, rows_v,
           need_v, sem):
    wid = lax.axis_index("s") * 2 + lax.axis_index("c")

    @pl.when(wid < B)
    def _():
        b = wid
        i32 = jnp.int32
        iota16 = lax.iota(i32, 16)
        ones16 = jnp.ones((16,), i32)
        zeros16 = jnp.zeros((16,), i32)

        pltpu.sync_copy(mk_hbm.at[b], mk_v)

        def zero_hist(_=None):
            def zb(j, _c):
                hist_v[pl.ds(j * 16, 16)] = zeros16
                return 0
            lax.fori_loop(0, HSIZE // 16, zb, 0)

        def suffix_scan():
            def sb(jj, cs):
                j = 127 - jj
                h = hist_v[pl.ds(j * 16, 16)]
                sfx = lax.rev(jnp.cumsum(lax.rev(h, (0,)), axis=0), (0,)) + cs
                suf_v[pl.ds(j * 16, 16)] = sfx
                return cs + jnp.sum(h)
            lax.fori_loop(0, 128, sb, i32(0))

        def count_ge(target):
            def cb(j, cnt):
                s = suf_v[pl.ds(j * 16, 16)]
                return cnt + jnp.sum((s >= target).astype(i32))
            return lax.fori_loop(0, 128, cb, i32(0))

        def at(ref, i):
            return jnp.max(plsc.load_gather(ref, [jnp.full((16,), i, i32)]))

        zero_hist()

        def h1(j, _c):
            v = mk_v[pl.ds(j * 16, 16)]
            bkt = jnp.where(v == 0, i32(2048), lax.shift_right_logical(v, 11))
            plsc.addupdate_scatter(hist_v, [bkt], ones16)
            return 0
        lax.fori_loop(0, NBLK, h1, 0)

        suffix_scan()
        n_valid = at(suf_v, i32(0))
        need = jnp.minimum(n_valid, i32(MAX_DET))
        t1 = count_ge(need) - 1
        s_t1 = at(suf_v, t1)
        h_t1 = at(hist_v, t1)
        n_ab1 = s_t1 - h_t1

        zero_hist()

        def h2(j, _c):
            v = mk_v[pl.ds(j * 16, 16)]
            msk = (lax.shift_right_logical(v, 11) == t1) & (v >= 1)
            plsc.addupdate_scatter(hist_v, [v & 2047], ones16, mask=msk)
            return 0
        lax.fori_loop(0, NBLK, h2, 0)

        suffix_scan()
        need2 = need - n_ab1
        t2 = count_ge(need2) - 1
        s_t2 = at(suf_v, t2)
        h_t2 = at(hist_v, t2)
        n_above = n_ab1 + (s_t2 - h_t2)
        need_eq = need - n_above
        kt = t1 * 2048 + t2

        def zero384(ref):
            for j in range(PAD_DET // 16):
                ref[pl.ds(j * 16, 16)] = zeros16
        zero384(gt_v)
        zero384(eq_v)
        zero384(srt_v)

        def comp(j, carry):
            og, oe = carry
            v = mk_v[pl.ds(j * 16, 16)]
            gidx = j * 16 + iota16
            mgt = v > kt
            cgt = jnp.cumsum(mgt.astype(i32), axis=0)
            plsc.store_scatter(gt_v, [og + cgt - 1], gidx, mask=mgt)
            og = og + jnp.sum(mgt.astype(i32))
            meq = (v == kt) & (v >= 1)
            ceq = jnp.cumsum(meq.astype(i32), axis=0)
            meq2 = meq & ((oe + ceq) <= need_eq)
            plsc.store_scatter(eq_v, [oe + ceq - 1], gidx, mask=meq2)
            oe = jnp.minimum(oe + jnp.sum(meq.astype(i32)), need_eq)
            return og, oe
        og, oe = lax.fori_loop(0, NBLK, comp, (i32(0), i32(0)))

        def kload(j, _c):
            lane = j * 16 + iota16
            gi = gt_v[pl.ds(j * 16, 16)]
            kk = plsc.load_gather(mk_v, [gi], mask=lane < og)
            keys_v[pl.ds(j * 16, 16)] = jnp.where(lane < og, kk, i32(-1))
            return 0
        lax.fori_loop(0, PAD_DET // 16, kload, 0)

        def rank(i, _c):
            ki = plsc.load_gather(keys_v, [jnp.full((16,), i, i32)])
            ii = plsc.load_gather(gt_v, [jnp.full((16,), i, i32)])

            def inner(j, cnt):
                lane = j * 16 + iota16
                kv = keys_v[pl.ds(j * 16, 16)]
                iv = gt_v[pl.ds(j * 16, 16)]
                c = (lane < og) & ((kv > ki) | ((kv == ki) & (iv < ii)))
                return cnt + c.astype(i32)
            cntv = lax.fori_loop(0, PAD_DET // 16, inner, zeros16)
            r = jnp.sum(cntv)
            plsc.store_scatter(srt_v, [jnp.full((16,), r, i32)], ii,
                               mask=iota16 == 0)
            return 0
        lax.fori_loop(0, og, rank, 0)

        def mrg(j, _c):
            lane = j * 16 + iota16
            e = eq_v[pl.ds(j * 16, 16)]
            plsc.store_scatter(srt_v, [og + lane], e, mask=lane < oe)
            return 0
        lax.fori_loop(0, PAD_DET // 16, mrg, 0)

        base = b * N

        def gi_build(r, _c):
            for q in range(8):
                srows = plsc.load_gather(
                    srt_v, [jnp.full((16,), 16 * r + 2 * q, i32)
                            + lax.shift_right_logical(iota16, 3)])
                gidx_v[r, pl.ds(q * 16, 16)] = (
                    (srows + base) * 8 + (iota16 & 7))
            return 0
        lax.fori_loop(0, PAD_DET // 16, gi_build, 0)

        def gfire(r, _c):
            pltpu.async_copy(pay_hbm.at[gidx_v.at[r]],
                             rows_v.at[pl.ds(r * 128, 128)], sem)
            return 0
        lax.fori_loop(0, PAD_DET // 16, gfire, 0)
        pltpu.make_async_copy(pay_hbm.at[pl.ds(0, PAD_DET * 8)], rows_v,
                              sem).wait()

        pltpu.sync_copy(rows_v, rows_out.at[b])
        need_v[...] = jnp.full((16,), need, i32)
        pltpu.sync_copy(need_v, needs_out.at[b])


def _k3_body(rows_ref, needs_ref, tgt_ref, tlen_ref,
             pb_ref, plab_ref, ps_ref, tb_ref, tlab_ref, ts_ref, iou_ref):
    rows = rows_ref[:, :PAD_COL, :]
    x1 = rows[:, :, 0]
    y1 = rows[:, :, 1]
    x2 = rows[:, :, 2]
    y2 = rows[:, :, 3]
    scores = rows[:, :MAX_DET, 4]
    labels = rows[:, :MAX_DET, 5]
    need = needs_ref[:, 0:1]
    area = jnp.clip(x2 - x1, 0.0, None) * jnp.clip(y2 - y1, 0.0, None)

    for ci in range(PAD_COL // 8):
        sl = slice(ci * 8, ci * 8 + 8)
        xi1 = x1[:, sl][:, :, None]
        yi1 = y1[:, sl][:, :, None]
        xi2 = x2[:, sl][:, :, None]
        yi2 = y2[:, sl][:, :, None]
        ai = area[:, sl][:, :, None]
        ltx = jnp.maximum(xi1, x1[:, None, :])
        lty = jnp.maximum(yi1, y1[:, None, :])
        rbx = jnp.minimum(xi2, x2[:, None, :])
        rby = jnp.minimum(yi2, y2[:, None, :])
        inter = jnp.clip(rbx - ltx, 0.0, None) * jnp.clip(rby - lty, 0.0, None)
        union = ai + area[:, None, :] - inter
        iou_ref[:, sl, :] = inter / (union + 1e-9)

    col = lax.broadcasted_iota(jnp.int32, (B, PAD_COL), 1)
    keep0 = (col < need).astype(jnp.float32)

    def nms_step(i, keep):
        row = iou_ref[:, pl.ds(i, 1), :][:, 0, :]
        cond = (row > NMS_T) & (keep > 0.0) & (col < i)
        sup = jnp.max(jnp.where(cond, 1.0, 0.0), axis=-1, keepdims=True)
        return jnp.where((col == i) & (sup > 0.0), 0.0, keep)

    keep = lax.fori_loop(0, MAX_DET, nms_step, keep0, unroll=False)
    keepf = keep[:, :MAX_DET]

    pb_ref[...] = jnp.stack(
        [x1[:, :MAX_DET] * keepf, y1[:, :MAX_DET] * keepf,
         x2[:, :MAX_DET] * keepf, y2[:, :MAX_DET] * keepf], axis=-1)
    ps_ref[...] = scores * keepf
    plab_ref[...] = (labels * keepf).astype(jnp.int32)

    tgt = tgt_ref[...]
    tlen = tlen_ref[...]
    tmf = (lax.broadcasted_iota(jnp.int32, tgt.shape[:2], 1) < tlen
           ).astype(jnp.float32)
    tcx = tgt[:, :, 0] * tmf
    tcy = tgt[:, :, 1] * tmf
    tw = tgt[:, :, 2] * tmf
    th = tgt[:, :, 3] * tmf
    tx1 = tcx - tw / 2.0
    ty1 = tcy - th / 2.0
    tb_ref[...] = jnp.stack([tx1, ty1, tx1 + tw, ty1 + th], axis=-1)
    ts_ref[...] = tgt[:, :, 4] * tmf
    tlab_ref[...] = (tgt[:, :, 5] * tmf).astype(jnp.int32)


def _k3(rows, needs, targets, tlen2d):
    T = targets.shape[1]
    return pl.pallas_call(
        _k3_body,
        out_shape=[
            jax.ShapeDtypeStruct((B, MAX_DET, 4), jnp.float32),
            jax.ShapeDtypeStruct((B, MAX_DET), jnp.int32),
            jax.ShapeDtypeStruct((B, MAX_DET), jnp.float32),
            jax.ShapeDtypeStruct((B, T, 4), jnp.float32),
            jax.ShapeDtypeStruct((B, T), jnp.int32),
            jax.ShapeDtypeStruct((B, T), jnp.float32),
        ],
        scratch_shapes=[pltpu.VMEM((B, PAD_COL, PAD_COL), jnp.float32)],
    )(rows, needs, targets, tlen2d)


def kernel(logits, targets, target_lengths):
    lflat3 = logits.reshape(NCHUNK, CHUNK, 5 + C)
    pay3, mkey3 = _k1(lflat3)
    pay = pay3.reshape(B * N * 8)
    mkey2d = mkey3.reshape(B, N)
    rows_flat, needs = _k2_sc(mkey2d, pay)
    rows = rows_flat.reshape(B, PAD_DET, 8)
    tlen2d = target_lengths.reshape(B, 1)
    return tuple(_k3(rows, needs, targets, tlen2d))
